# Initial kernel scaffold; baseline (speedup 1.0000x reference)
import functools

import jax
import jax.numpy as jnp
from jax import lax
from jax.experimental import pallas as pl
from jax.experimental.pallas import tpu as pltpu
from jax.experimental.pallas import tpu_sc as plsc

N = 10000
E = 320000
D = 128

NC = 2
NS = 16
L = 16
NW = NC * NS
EPW = E // NW
C = 80
NCH = EPW // C
RPT = N // NS
ZR = 125


def _mm_body(x_ref, w_ref, o_ref):
    o_ref[...] = jnp.dot(x_ref[...], w_ref[...],
                         preferred_element_type=jnp.float32)


def _combine_body(p_ref, o_ref):
    o_ref[...] = jnp.maximum(p_ref[0] + p_ref[1], 0.0)


def _sc_body(h_hbm, src_hbm, dst_hbm, vals_hbm, out_hbm,
             src_v, dst_v, vals_v, rows_v, zero_v, acc_sh, sem):
    cid = lax.axis_index("c")
    sid = lax.axis_index("s")
    wid = cid * NS + sid

    def _zrow(i, carry):
        for t in range(D // L):
            zero_v[i, pl.ds(t * L, L)] = jnp.zeros((L,), jnp.float32)
        return carry
    lax.fori_loop(0, ZR, _zrow, 0)
    for b in range(RPT // ZR):
        pltpu.sync_copy(zero_v, acc_sh.at[pl.ds(sid * RPT + b * ZR, ZR)])
    plsc.subcore_barrier()

    pltpu.sync_copy(src_hbm.at[wid], src_v)
    pltpu.sync_copy(dst_hbm.at[wid], dst_v)
    pltpu.sync_copy(vals_hbm.at[wid], vals_v)

    def _chunk(j, carry):
        pltpu.async_copy(h_hbm.at[src_v.at[j]], rows_v, sem).wait()

        def _edge(e, c2):
            vi = jnp.full((L,), e, dtype=jnp.int32)
            vj = jnp.full((L,), j, dtype=jnp.int32)
            v = plsc.load_gather(vals_v, [vj, vi])
            for t in range(D // L):
                rows_v[e, pl.ds(t * L, L)] = rows_v[e, pl.ds(t * L, L)] * v
            return c2
        lax.fori_loop(0, C, _edge, 0)

        pltpu.sync_copy(rows_v, acc_sh.at[dst_v.at[j]], add=True)
        return carry
    lax.fori_loop(0, NCH, _chunk, 0)

    plsc.subcore_barrier()
    pltpu.sync_copy(acc_sh.at[pl.ds(sid * RPT, RPT)],
                    out_hbm.at[cid, pl.ds(sid * RPT, RPT)])


_sc_call = functools.partial(
    pl.kernel,
    out_type=jax.ShapeDtypeStruct((NC, N, D), jnp.float32),
    mesh=plsc.VectorSubcoreMesh(core_axis_name="c", subcore_axis_name="s"),
    scratch_types=[
        pltpu.VMEM((NCH, C), jnp.int32),
        pltpu.VMEM((NCH, C), jnp.int32),
        pltpu.VMEM((NCH, C), jnp.float32),
        pltpu.VMEM((C, D), jnp.float32),
        pltpu.VMEM((ZR, D), jnp.float32),
        pltpu.VMEM_SHARED((N, D), jnp.float32),
        pltpu.SemaphoreType.DMA,
    ],
)(_sc_body)


def kernel(x, W, edge_index, adj_vals):
    h = pl.pallas_call(
        _mm_body,
        grid=(5,),
        in_specs=[pl.BlockSpec((N // 5, D), lambda i: (i, 0)),
                  pl.BlockSpec((D, D), lambda i: (0, 0))],
        out_specs=pl.BlockSpec((N // 5, D), lambda i: (i, 0)),
        out_shape=jax.ShapeDtypeStruct((N, D), jnp.float32),
    )(x, W)

    src_r = edge_index[1].reshape(NW, NCH, C)
    dst_r = edge_index[0].reshape(NW, NCH, C)
    vals_r = adj_vals.reshape(NW, NCH, C)

    partials = _sc_call(h, src_r, dst_r, vals_r)

    out = pl.pallas_call(
        _combine_body,
        grid=(5,),
        in_specs=[pl.BlockSpec((NC, N // 5, D), lambda i: (0, i, 0))],
        out_specs=pl.BlockSpec((N // 5, D), lambda i: (i, 0)),
        out_shape=jax.ShapeDtypeStruct((N, D), jnp.float32),
    )(partials)
    return out


# speedup vs baseline: 3.0110x; 3.0110x over previous
import functools

import jax
import jax.numpy as jnp
from jax import lax
from jax.experimental import pallas as pl
from jax.experimental.pallas import tpu as pltpu
from jax.experimental.pallas import tpu_sc as plsc

N = 10000
E = 320000
D = 128

NC = 2
NS = 16
L = 16
NW = NC * NS
EPW = 10240
EP = NW * EPW
C = 128
NCH = EPW // C
NB = 16
NBLK = NCH // NB
NP = 10240
RPT = NP // NS
ZR = 128

_BCAST_DNUMS = lax.GatherDimensionNumbers(
    offset_dims=(), collapsed_slice_dims=(0,), start_index_map=(0,))


def _lane_bcast(v16, lane):
    idx = jnp.full((L, 1), lane, dtype=jnp.int32)
    return lax.gather(v16, idx, _BCAST_DNUMS, slice_sizes=(1,),
                      mode=lax.GatherScatterMode.PROMISE_IN_BOUNDS)


def _mm_body(x_ref, w_ref, o_ref):
    o_ref[...] = jnp.dot(x_ref[...], w_ref[...],
                         preferred_element_type=jnp.float32)


def _combine_body(p_ref, o_ref):
    o_ref[...] = jnp.maximum(p_ref[0] + p_ref[1], 0.0)


def _sc_body(h_hbm, src_hbm, dst_hbm, vals_hbm, out_hbm,
             src_v, dst_v, vals_v, rows_v, acc_sh, sem):
    cid = lax.axis_index("c")
    sid = lax.axis_index("s")
    wid = cid * NS + sid

    def _zrow(i, carry):
        for t in range(D // L):
            rows_v[i, pl.ds(t * L, L)] = jnp.zeros((L,), jnp.float32)
        return carry
    lax.fori_loop(0, ZR, _zrow, 0)
    for b in range(RPT // ZR):
        pltpu.sync_copy(rows_v.at[pl.ds(0, ZR)],
                        acc_sh.at[pl.ds(sid * RPT + b * ZR, ZR)])
    plsc.subcore_barrier()

    for blk in range(NBLK):
        pltpu.sync_copy(src_hbm.at[wid, pl.ds(blk * NB, NB)], src_v)
        pltpu.sync_copy(dst_hbm.at[wid, pl.ds(blk * NB, NB)], dst_v)
        pltpu.sync_copy(vals_hbm.at[wid, pl.ds(blk * NB, NB)], vals_v)

        def _chunk(j, carry):
            pltpu.async_copy(h_hbm.at[src_v.at[j]], rows_v, sem).wait()

            def _edge(e, c2):
                v16 = vals_v[j, pl.ds(e & ~(L - 1), L)]
                v = _lane_bcast(v16, e & (L - 1))
                for t in range(D // L):
                    rows_v[e, pl.ds(t * L, L)] = rows_v[e, pl.ds(t * L, L)] * v
                return c2
            lax.fori_loop(0, C, _edge, 0)

            pltpu.sync_copy(rows_v, acc_sh.at[dst_v.at[j]], add=True)
            return carry
        lax.fori_loop(0, NB, _chunk, 0)

    plsc.subcore_barrier()
    pltpu.sync_copy(acc_sh.at[pl.ds(sid * RPT, RPT)],
                    out_hbm.at[cid, pl.ds(sid * RPT, RPT)])


_sc_call = functools.partial(
    pl.kernel,
    out_type=jax.ShapeDtypeStruct((NC, NP, D), jnp.float32),
    mesh=plsc.VectorSubcoreMesh(core_axis_name="c", subcore_axis_name="s"),
    scratch_types=[
        pltpu.VMEM((NB, C), jnp.int32),
        pltpu.VMEM((NB, C), jnp.int32),
        pltpu.VMEM((NB, C), jnp.float32),
        pltpu.VMEM((C, D), jnp.float32),
        pltpu.VMEM_SHARED((NP, D), jnp.float32),
        pltpu.SemaphoreType.DMA,
    ],
)(_sc_body)


def kernel(x, W, edge_index, adj_vals):
    h = pl.pallas_call(
        _mm_body,
        grid=(5,),
        in_specs=[pl.BlockSpec((N // 5, D), lambda i: (i, 0)),
                  pl.BlockSpec((D, D), lambda i: (0, 0))],
        out_specs=pl.BlockSpec((N // 5, D), lambda i: (i, 0)),
        out_shape=jax.ShapeDtypeStruct((N, D), jnp.float32),
    )(x, W)

    pad = EP - E
    zi = jnp.zeros((pad,), jnp.int32)
    src_r = jnp.concatenate([edge_index[1], zi]).reshape(NW, NCH, C)
    dst_r = jnp.concatenate([edge_index[0], zi]).reshape(NW, NCH, C)
    vals_r = jnp.concatenate(
        [adj_vals, jnp.zeros((pad,), jnp.float32)]).reshape(NW, NCH, C)

    partials = _sc_call(h, src_r, dst_r, vals_r)

    out = pl.pallas_call(
        _combine_body,
        grid=(5,),
        in_specs=[pl.BlockSpec((NC, N // 5, D), lambda i: (0, i, 0))],
        out_specs=pl.BlockSpec((N // 5, D), lambda i: (i, 0)),
        out_shape=jax.ShapeDtypeStruct((N, D), jnp.float32),
    )(partials)
    return out


# speedup vs baseline: 8.0491x; 2.6733x over previous
import functools

import jax
import jax.numpy as jnp
from jax import lax
from jax.experimental import pallas as pl
from jax.experimental.pallas import tpu as pltpu
from jax.experimental.pallas import tpu_sc as plsc

N = 10000
E = 320000
D = 128

NC = 2
NS = 16
L = 16
NW = NC * NS
EPW = 10240
EP = NW * EPW
C = 64
NCH = EPW // C
NB = 32
NBLK = NCH // NB
NP = 10240
RPT = NP // NS
ZR = 64

_BCAST_DNUMS = lax.GatherDimensionNumbers(
    offset_dims=(), collapsed_slice_dims=(0,), start_index_map=(0,))


def _lane_bcast(v16, lane):
    idx = jnp.full((L, 1), lane, dtype=jnp.int32)
    return lax.gather(v16, idx, _BCAST_DNUMS, slice_sizes=(1,),
                      mode=lax.GatherScatterMode.PROMISE_IN_BOUNDS)


def _mm_body(x_ref, w_ref, o_ref):
    o_ref[...] = jnp.dot(x_ref[...], w_ref[...],
                         preferred_element_type=jnp.float32)


def _combine_body(p_ref, o_ref):
    o_ref[...] = jnp.maximum(p_ref[0] + p_ref[1], 0.0)


def _sc_body(h_hbm, src_hbm, dst_hbm, vals_hbm, out_hbm,
             src_v, dst_v, vals_v, rows_a, rows_b, acc_sh, sem_a, sem_b):
    cid = lax.axis_index("c")
    sid = lax.axis_index("s")
    wid = cid * NS + sid

    def _zrow(i, carry):
        for t in range(D // L):
            rows_a[i, pl.ds(t * L, L)] = jnp.zeros((L,), jnp.float32)
        return carry
    lax.fori_loop(0, ZR, _zrow, 0)
    for b in range(RPT // ZR):
        pltpu.sync_copy(rows_a.at[pl.ds(0, ZR)],
                        acc_sh.at[pl.ds(sid * RPT + b * ZR, ZR)])
    plsc.subcore_barrier()

    def _gather(j, buf, sem):
        pltpu.async_copy(h_hbm.at[src_v.at[j]], buf, sem)

    def _gwait(buf, sem):
        pltpu.make_async_copy(h_hbm.at[src_v.at[0]], buf, sem).wait()

    def _scale_scatter(j, buf):
        def _edge(e, c2):
            v16 = vals_v[j, pl.ds(e & ~(L - 1), L)]
            v = _lane_bcast(v16, e & (L - 1))
            for t in range(D // L):
                buf[e, pl.ds(t * L, L)] = buf[e, pl.ds(t * L, L)] * v
            return c2
        lax.fori_loop(0, C, _edge, 0)
        pltpu.sync_copy(buf, acc_sh.at[dst_v.at[j]], add=True)

    for blk in range(NBLK):
        pltpu.sync_copy(src_hbm.at[wid, pl.ds(blk * NB, NB)], src_v)
        pltpu.sync_copy(dst_hbm.at[wid, pl.ds(blk * NB, NB)], dst_v)
        pltpu.sync_copy(vals_hbm.at[wid, pl.ds(blk * NB, NB)], vals_v)

        _gather(0, rows_a, sem_a)

        def _pair(p, carry):
            j0 = 2 * p
            _gwait(rows_a, sem_a)
            _gather(j0 + 1, rows_b, sem_b)
            _scale_scatter(j0, rows_a)
            _gwait(rows_b, sem_b)

            @pl.when(p < NB // 2 - 1)
            def _():
                _gather(j0 + 2, rows_a, sem_a)
            _scale_scatter(j0 + 1, rows_b)
            return carry
        lax.fori_loop(0, NB // 2, _pair, 0)

    plsc.subcore_barrier()
    pltpu.sync_copy(acc_sh.at[pl.ds(sid * RPT, RPT)],
                    out_hbm.at[cid, pl.ds(sid * RPT, RPT)])


_sc_call = functools.partial(
    pl.kernel,
    out_type=jax.ShapeDtypeStruct((NC, NP, D), jnp.float32),
    mesh=plsc.VectorSubcoreMesh(core_axis_name="c", subcore_axis_name="s"),
    scratch_types=[
        pltpu.VMEM((NB, C), jnp.int32),
        pltpu.VMEM((NB, C), jnp.int32),
        pltpu.VMEM((NB, C), jnp.float32),
        pltpu.VMEM((C, D), jnp.float32),
        pltpu.VMEM((C, D), jnp.float32),
        pltpu.VMEM_SHARED((NP, D), jnp.float32),
        pltpu.SemaphoreType.DMA,
        pltpu.SemaphoreType.DMA,
    ],
)(_sc_body)


def kernel(x, W, edge_index, adj_vals):
    h = pl.pallas_call(
        _mm_body,
        grid=(5,),
        in_specs=[pl.BlockSpec((N // 5, D), lambda i: (i, 0)),
                  pl.BlockSpec((D, D), lambda i: (0, 0))],
        out_specs=pl.BlockSpec((N // 5, D), lambda i: (i, 0)),
        out_shape=jax.ShapeDtypeStruct((N, D), jnp.float32),
    )(x, W)

    pad = EP - E
    pi = jnp.arange(pad, dtype=jnp.int32)
    src_r = jnp.concatenate(
        [edge_index[1], pi % N]).reshape(NW, NCH, C)
    dst_r = jnp.concatenate(
        [edge_index[0], N + pi % (NP - N)]).reshape(NW, NCH, C)
    vals_r = jnp.concatenate(
        [adj_vals, jnp.zeros((pad,), jnp.float32)]).reshape(NW, NCH, C)

    partials = _sc_call(h, src_r, dst_r, vals_r)

    out = pl.pallas_call(
        _combine_body,
        grid=(5,),
        in_specs=[pl.BlockSpec((NC, N // 5, D), lambda i: (0, i, 0))],
        out_specs=pl.BlockSpec((N // 5, D), lambda i: (i, 0)),
        out_shape=jax.ShapeDtypeStruct((N, D), jnp.float32),
    )(partials)
    return out


# speedup vs baseline: 8.4992x; 1.0559x over previous
import functools

import jax
import jax.numpy as jnp
from jax import lax
from jax.experimental import pallas as pl
from jax.experimental.pallas import tpu as pltpu
from jax.experimental.pallas import tpu_sc as plsc

N = 10000
E = 320000
D = 128

NC = 2
NS = 16
L = 16
NW = NC * NS
EPW = 10240
EP = NW * EPW
C = 64
NCH = EPW // C
NB = 32
NBLK = NCH // NB
NP = 10240
RPT = NP // NS
ZR = 64

_BCAST_DNUMS = lax.GatherDimensionNumbers(
    offset_dims=(), collapsed_slice_dims=(0,), start_index_map=(0,))


def _lane_bcast(v16, lane):
    idx = jnp.full((L, 1), lane, dtype=jnp.int32)
    return lax.gather(v16, idx, _BCAST_DNUMS, slice_sizes=(1,),
                      mode=lax.GatherScatterMode.PROMISE_IN_BOUNDS)


def _mm_body(x_ref, w_ref, o_ref):
    o_ref[...] = jnp.dot(x_ref[...], w_ref[...],
                         preferred_element_type=jnp.float32)


def _combine_body(p_ref, o_ref):
    o_ref[...] = jnp.maximum(p_ref[0] + p_ref[1], 0.0)


def _sc_body(h_hbm, src_hbm, dst_hbm, vals_hbm, out_hbm,
             src_v, dst_v, vals_v, rows_a, rows_b, acc_sh,
             sem_a, sem_b, sem_sa, sem_sb):
    cid = lax.axis_index("c")
    sid = lax.axis_index("s")
    wid = cid * NS + sid

    def _zrow(i, carry):
        for t in range(D // L):
            rows_a[i, pl.ds(t * L, L)] = jnp.zeros((L,), jnp.float32)
        return carry
    lax.fori_loop(0, ZR, _zrow, 0)
    for b in range(RPT // ZR):
        pltpu.sync_copy(rows_a.at[pl.ds(0, ZR)],
                        acc_sh.at[pl.ds(sid * RPT + b * ZR, ZR)])
    plsc.subcore_barrier()

    def _gather(j, buf, sem):
        pltpu.async_copy(h_hbm.at[src_v.at[j]], buf, sem)

    def _gwait(buf, sem):
        pltpu.make_async_copy(h_hbm.at[src_v.at[0]], buf, sem).wait()

    def _sfire(j, buf, sem):
        pltpu.async_copy(buf, acc_sh.at[dst_v.at[j]], sem, add=True)

    def _swait(buf, sem):
        pltpu.make_async_copy(buf, acc_sh.at[dst_v.at[0]], sem).wait()

    def _scale(j, buf):
        def _grp(g, c2):
            base = g * L
            v16 = vals_v[j, pl.ds(base, L)]
            for lane in range(L):
                v = _lane_bcast(v16, lane)
                e = base + lane
                for t in range(D // L):
                    buf[e, pl.ds(t * L, L)] = buf[e, pl.ds(t * L, L)] * v
            return c2
        lax.fori_loop(0, C // L, _grp, 0)

    for blk in range(NBLK):
        pltpu.sync_copy(src_hbm.at[wid, pl.ds(blk * NB, NB)], src_v)
        pltpu.sync_copy(dst_hbm.at[wid, pl.ds(blk * NB, NB)], dst_v)
        pltpu.sync_copy(vals_hbm.at[wid, pl.ds(blk * NB, NB)], vals_v)

        _gather(0, rows_a, sem_a)

        def _pair(p, carry):
            j0 = 2 * p
            _gwait(rows_a, sem_a)

            @pl.when(p > 0)
            def _():
                _swait(rows_b, sem_sb)
            _gather(j0 + 1, rows_b, sem_b)
            _scale(j0, rows_a)
            _sfire(j0, rows_a, sem_sa)
            _gwait(rows_b, sem_b)
            _swait(rows_a, sem_sa)

            @pl.when(p < NB // 2 - 1)
            def _():
                _gather(j0 + 2, rows_a, sem_a)
            _scale(j0 + 1, rows_b)
            _sfire(j0 + 1, rows_b, sem_sb)
            return carry
        lax.fori_loop(0, NB // 2, _pair, 0)
        _swait(rows_b, sem_sb)

    plsc.subcore_barrier()
    pltpu.sync_copy(acc_sh.at[pl.ds(sid * RPT, RPT)],
                    out_hbm.at[cid, pl.ds(sid * RPT, RPT)])


_sc_call = functools.partial(
    pl.kernel,
    out_type=jax.ShapeDtypeStruct((NC, NP, D), jnp.float32),
    mesh=plsc.VectorSubcoreMesh(core_axis_name="c", subcore_axis_name="s"),
    scratch_types=[
        pltpu.VMEM((NB, C), jnp.int32),
        pltpu.VMEM((NB, C), jnp.int32),
        pltpu.VMEM((NB, C), jnp.float32),
        pltpu.VMEM((C, D), jnp.float32),
        pltpu.VMEM((C, D), jnp.float32),
        pltpu.VMEM_SHARED((NP, D), jnp.float32),
        pltpu.SemaphoreType.DMA,
        pltpu.SemaphoreType.DMA,
        pltpu.SemaphoreType.DMA,
        pltpu.SemaphoreType.DMA,
    ],
)(_sc_body)


def kernel(x, W, edge_index, adj_vals):
    h = pl.pallas_call(
        _mm_body,
        grid=(5,),
        in_specs=[pl.BlockSpec((N // 5, D), lambda i: (i, 0)),
                  pl.BlockSpec((D, D), lambda i: (0, 0))],
        out_specs=pl.BlockSpec((N // 5, D), lambda i: (i, 0)),
        out_shape=jax.ShapeDtypeStruct((N, D), jnp.float32),
    )(x, W)

    pad = EP - E
    pi = jnp.arange(pad, dtype=jnp.int32)
    src_r = jnp.concatenate(
        [edge_index[1], pi % N]).reshape(NW, NCH, C)
    dst_r = jnp.concatenate(
        [edge_index[0], N + pi % (NP - N)]).reshape(NW, NCH, C)
    vals_r = jnp.concatenate(
        [adj_vals, jnp.zeros((pad,), jnp.float32)]).reshape(NW, NCH, C)

    partials = _sc_call(h, src_r, dst_r, vals_r)

    out = pl.pallas_call(
        _combine_body,
        grid=(5,),
        in_specs=[pl.BlockSpec((NC, N // 5, D), lambda i: (0, i, 0))],
        out_specs=pl.BlockSpec((N // 5, D), lambda i: (i, 0)),
        out_shape=jax.ShapeDtypeStruct((N, D), jnp.float32),
    )(partials)
    return out


# speedup vs baseline: 10.8089x; 1.2718x over previous
import functools

import jax
import jax.numpy as jnp
from jax import lax
from jax.experimental import pallas as pl
from jax.experimental.pallas import tpu as pltpu
from jax.experimental.pallas import tpu_sc as plsc

N = 10000
E = 320000
D = 128

NC = 2
NS = 16
L = 16
NW = NC * NS
EPW = 10240
EP = NW * EPW
C = 16
NCH = EPW // C
NB = 64
NBLK = NCH // NB
NG = 8
NO = NB // NG
NP = 10240
RPT = NP // NS
ZR = 16

_BCAST_DNUMS = lax.GatherDimensionNumbers(
    offset_dims=(), collapsed_slice_dims=(0,), start_index_map=(0,))


def _lane_bcast(v16, lane):
    idx = jnp.full((L, 1), lane, dtype=jnp.int32)
    return lax.gather(v16, idx, _BCAST_DNUMS, slice_sizes=(1,),
                      mode=lax.GatherScatterMode.PROMISE_IN_BOUNDS)


def _mm_body(x_ref, w_ref, o_ref):
    o_ref[...] = jnp.dot(x_ref[...], w_ref[...],
                         preferred_element_type=jnp.float32)


def _combine_body(p_ref, o_ref):
    o_ref[...] = jnp.maximum(p_ref[0] + p_ref[1], 0.0)


def _sc_body(h_hbm, src_hbm, dst_hbm, vals_hbm, out_hbm,
             src_v, dst_v, vals_v,
             g0, g1, g2, g3, g4, g5, g6, g7, acc_sh,
             mg0, mg1, mg2, mg3, mg4, mg5, mg6, mg7,
             ms0, ms1, ms2, ms3, ms4, ms5, ms6, ms7):
    cid = lax.axis_index("c")
    sid = lax.axis_index("s")
    wid = cid * NS + sid
    gbufs = (g0, g1, g2, g3, g4, g5, g6, g7)
    gsems = (mg0, mg1, mg2, mg3, mg4, mg5, mg6, mg7)
    ssems = (ms0, ms1, ms2, ms3, ms4, ms5, ms6, ms7)

    def _zrow(i, carry):
        for t in range(D // L):
            g0[i, pl.ds(t * L, L)] = jnp.zeros((L,), jnp.float32)
        return carry
    lax.fori_loop(0, ZR, _zrow, 0)
    for b in range(RPT // ZR):
        pltpu.sync_copy(g0, acc_sh.at[pl.ds(sid * RPT + b * ZR, ZR)])
    plsc.subcore_barrier()

    def _gather(j, q):
        pltpu.async_copy(h_hbm.at[src_v.at[j]], gbufs[q], gsems[q])

    def _gwait(q):
        pltpu.make_async_copy(h_hbm.at[src_v.at[0]], gbufs[q],
                              gsems[q]).wait()

    def _sfire(j, q):
        pltpu.async_copy(gbufs[q], acc_sh.at[dst_v.at[j]], ssems[q],
                         add=True)

    def _swait(q):
        pltpu.make_async_copy(gbufs[q], acc_sh.at[dst_v.at[0]],
                              ssems[q]).wait()

    def _scale(j, q):
        gb = gbufs[q]
        v16 = vals_v[j, :]

        def _e(lane, c2):
            v = _lane_bcast(v16, lane)
            for t in range(D // L):
                gb[lane, pl.ds(t * L, L)] = gb[lane, pl.ds(t * L, L)] * v
            return c2
        lax.fori_loop(0, C, _e, 0)

    for blk in range(NBLK):
        pltpu.sync_copy(src_hbm.at[wid, pl.ds(blk * NB, NB)], src_v)
        pltpu.sync_copy(dst_hbm.at[wid, pl.ds(blk * NB, NB)], dst_v)
        pltpu.sync_copy(vals_hbm.at[wid, pl.ds(blk * NB, NB)], vals_v)

        for q in range(NG):
            _gather(q, q)

        def _oct(p, carry):
            j0 = NG * p
            for q in range(NG):
                j = j0 + q
                _gwait(q)
                _scale(j, q)
                _sfire(j, q)
                if q >= 2:
                    qq = q - 2

                    @pl.when(p < NO - 1)
                    def _():
                        _swait(qq)
                        _gather(j0 + NG + qq, qq)

            @pl.when(p < NO - 1)
            def _():
                for qq in (NG - 2, NG - 1):
                    _swait(qq)
                    _gather(j0 + NG + qq, qq)
            return carry
        lax.fori_loop(0, NO, _oct, 0)
        for q in range(NG):
            _swait(q)

    plsc.subcore_barrier()
    pltpu.sync_copy(acc_sh.at[pl.ds(sid * RPT, RPT)],
                    out_hbm.at[cid, pl.ds(sid * RPT, RPT)])


_sc_call = functools.partial(
    pl.kernel,
    out_type=jax.ShapeDtypeStruct((NC, NP, D), jnp.float32),
    mesh=plsc.VectorSubcoreMesh(core_axis_name="c", subcore_axis_name="s"),
    compiler_params=pltpu.CompilerParams(use_tc_tiling_on_sc=False),
    scratch_types=(
        [pltpu.VMEM((NB, C), jnp.int32),
         pltpu.VMEM((NB, C), jnp.int32),
         pltpu.VMEM((NB, C), jnp.float32)]
        + [pltpu.VMEM((C, D), jnp.float32) for _ in range(NG)]
        + [pltpu.VMEM_SHARED((NP, D), jnp.float32)]
        + [pltpu.SemaphoreType.DMA for _ in range(2 * NG)]
    ),
)(_sc_body)


def kernel(x, W, edge_index, adj_vals):
    h = pl.pallas_call(
        _mm_body,
        grid=(5,),
        in_specs=[pl.BlockSpec((N // 5, D), lambda i: (i, 0)),
                  pl.BlockSpec((D, D), lambda i: (0, 0))],
        out_specs=pl.BlockSpec((N // 5, D), lambda i: (i, 0)),
        out_shape=jax.ShapeDtypeStruct((N, D), jnp.float32),
    )(x, W)

    pad = EP - E
    pi = jnp.arange(pad, dtype=jnp.int32)
    src_r = jnp.concatenate([edge_index[1], pi % N]).reshape(NW, NCH, C)
    dst_r = jnp.concatenate(
        [edge_index[0], N + pi % (NP - N)]).reshape(NW, NCH, C)
    vals_r = jnp.concatenate(
        [adj_vals, jnp.zeros((pad,), jnp.float32)]).reshape(NW, NCH, C)

    partials = _sc_call(h, src_r, dst_r, vals_r)

    out = pl.pallas_call(
        _combine_body,
        grid=(5,),
        in_specs=[pl.BlockSpec((NC, N // 5, D), lambda i: (0, i, 0))],
        out_specs=pl.BlockSpec((N // 5, D), lambda i: (i, 0)),
        out_shape=jax.ShapeDtypeStruct((N, D), jnp.float32),
    )(partials)
    return out


# speedup vs baseline: 10.9534x; 1.0134x over previous
import functools

import jax
import jax.numpy as jnp
from jax import lax
from jax.experimental import pallas as pl
from jax.experimental.pallas import tpu as pltpu
from jax.experimental.pallas import tpu_sc as plsc

N = 10000
E = 320000
D = 128

NC = 2
NS = 16
L = 16
NW = NC * NS
EPW = 10240
EP = NW * EPW
C = 16
NCH = EPW // C
NB = 64
NBLK = NCH // NB
NG = 8
NO = NB // NG
NP = 10240
RPT = NP // NS
ZR = 16

_BCAST_DNUMS = lax.GatherDimensionNumbers(
    offset_dims=(), collapsed_slice_dims=(0,), start_index_map=(0,))


def _lane_bcast(v16, lane):
    idx = jnp.full((L, 1), lane, dtype=jnp.int32)
    return lax.gather(v16, idx, _BCAST_DNUMS, slice_sizes=(1,),
                      mode=lax.GatherScatterMode.PROMISE_IN_BOUNDS)


def _mm_body(x_ref, w_ref, o_ref):
    o_ref[...] = jnp.dot(x_ref[...], w_ref[...],
                         preferred_element_type=jnp.float32)


def _combine_body(p_ref, o_ref):
    o_ref[...] = jnp.maximum(p_ref[0] + p_ref[1], 0.0)


def _sc_body(h_hbm, src_hbm, dst_hbm, vals_hbm, out_hbm,
             src_v, dst_v, vals_v,
             g0, g1, g2, g3, g4, g5, g6, g7, zb, acc_sh,
             mg0, mg1, mg2, mg3, mg4, mg5, mg6, mg7,
             ms0, ms1, ms2, ms3, ms4, ms5, ms6, ms7, mz):
    cid = lax.axis_index("c")
    sid = lax.axis_index("s")
    wid = cid * NS + sid
    gbufs = (g0, g1, g2, g3, g4, g5, g6, g7)
    gsems = (mg0, mg1, mg2, mg3, mg4, mg5, mg6, mg7)
    ssems = (ms0, ms1, ms2, ms3, ms4, ms5, ms6, ms7)

    def _gather(j, q):
        pltpu.async_copy(h_hbm.at[src_v.at[j]], gbufs[q], gsems[q])

    def _gwait(q):
        pltpu.make_async_copy(h_hbm.at[src_v.at[0]], gbufs[q],
                              gsems[q]).wait()

    def _sfire(j, q):
        pltpu.async_copy(gbufs[q], acc_sh.at[dst_v.at[j]], ssems[q],
                         add=True)

    def _swait(q):
        pltpu.make_async_copy(gbufs[q], acc_sh.at[dst_v.at[0]],
                              ssems[q]).wait()

    def _scale(j, q):
        gb = gbufs[q]
        v16 = vals_v[j, :]

        def _e(lane, c2):
            v = _lane_bcast(v16, lane)
            for t in range(D // L):
                gb[lane, pl.ds(t * L, L)] = gb[lane, pl.ds(t * L, L)] * v
            return c2
        lax.fori_loop(0, C, _e, 0)

    for blk in range(NBLK):
        pltpu.sync_copy(src_hbm.at[wid, pl.ds(blk * NB, NB)], src_v)
        pltpu.sync_copy(dst_hbm.at[wid, pl.ds(blk * NB, NB)], dst_v)
        pltpu.sync_copy(vals_hbm.at[wid, pl.ds(blk * NB, NB)], vals_v)

        for q in range(NG):
            _gather(q, q)

        if blk == 0:
            def _zrow(i, carry):
                for t in range(D // L):
                    zb[i, pl.ds(t * L, L)] = jnp.zeros((L,), jnp.float32)
                return carry
            lax.fori_loop(0, ZR, _zrow, 0)
            for b in range(RPT // ZR):
                pltpu.async_copy(
                    zb, acc_sh.at[pl.ds(sid * RPT + b * ZR, ZR)], mz)
            for b in range(RPT // ZR):
                pltpu.make_async_copy(
                    zb, acc_sh.at[pl.ds(sid * RPT, ZR)], mz).wait()
            plsc.subcore_barrier()

        def _oct(p, carry):
            j0 = NG * p
            for q in range(NG):
                j = j0 + q
                _gwait(q)
                _scale(j, q)
                _sfire(j, q)
                if q >= 2:
                    qq = q - 2

                    @pl.when(p < NO - 1)
                    def _():
                        _swait(qq)
                        _gather(j0 + NG + qq, qq)

            @pl.when(p < NO - 1)
            def _():
                for qq in (NG - 2, NG - 1):
                    _swait(qq)
                    _gather(j0 + NG + qq, qq)
            return carry
        lax.fori_loop(0, NO, _oct, 0)
        for q in range(NG):
            _swait(q)

    plsc.subcore_barrier()
    pltpu.sync_copy(acc_sh.at[pl.ds(sid * RPT, RPT)],
                    out_hbm.at[cid, pl.ds(sid * RPT, RPT)])


_sc_call = functools.partial(
    pl.kernel,
    out_type=jax.ShapeDtypeStruct((NC, NP, D), jnp.float32),
    mesh=plsc.VectorSubcoreMesh(core_axis_name="c", subcore_axis_name="s"),
    compiler_params=pltpu.CompilerParams(use_tc_tiling_on_sc=False),
    scratch_types=(
        [pltpu.VMEM((NB, C), jnp.int32),
         pltpu.VMEM((NB, C), jnp.int32),
         pltpu.VMEM((NB, C), jnp.float32)]
        + [pltpu.VMEM((C, D), jnp.float32) for _ in range(NG)]
        + [pltpu.VMEM((ZR, D), jnp.float32)]
        + [pltpu.VMEM_SHARED((NP, D), jnp.float32)]
        + [pltpu.SemaphoreType.DMA for _ in range(2 * NG + 1)]
    ),
)(_sc_body)


def kernel(x, W, edge_index, adj_vals):
    h = pl.pallas_call(
        _mm_body,
        grid=(5,),
        in_specs=[pl.BlockSpec((N // 5, D), lambda i: (i, 0)),
                  pl.BlockSpec((D, D), lambda i: (0, 0))],
        out_specs=pl.BlockSpec((N // 5, D), lambda i: (i, 0)),
        out_shape=jax.ShapeDtypeStruct((N, D), jnp.float32),
    )(x, W)

    pad = EP - E
    pi = jnp.arange(pad, dtype=jnp.int32)
    src_r = jnp.concatenate([edge_index[1], pi % N]).reshape(NW, NCH, C)
    dst_r = jnp.concatenate(
        [edge_index[0], N + pi % (NP - N)]).reshape(NW, NCH, C)
    vals_r = jnp.concatenate(
        [adj_vals, jnp.zeros((pad,), jnp.float32)]).reshape(NW, NCH, C)

    partials = _sc_call(h, src_r, dst_r, vals_r)

    out = pl.pallas_call(
        _combine_body,
        grid=(5,),
        in_specs=[pl.BlockSpec((NC, N // 5, D), lambda i: (0, i, 0))],
        out_specs=pl.BlockSpec((N // 5, D), lambda i: (i, 0)),
        out_shape=jax.ShapeDtypeStruct((N, D), jnp.float32),
    )(partials)
    return out


# speedup vs baseline: 11.7744x; 1.0750x over previous
import functools

import jax
import jax.numpy as jnp
from jax import lax
from jax.experimental import pallas as pl
from jax.experimental.pallas import tpu as pltpu
from jax.experimental.pallas import tpu_sc as plsc

N = 10000
E = 320000
D = 128

NC = 2
NS = 16
L = 16
NW = NC * NS
EPW = 10240
EP = NW * EPW
C = 16
NCH = EPW // C
NB = 128
NBLK = NCH // NB
NG = 8
NO = NB // NG
NP = 10240
RPT = NP // NS
ZR = 16

_BCAST_DNUMS = lax.GatherDimensionNumbers(
    offset_dims=(), collapsed_slice_dims=(0,), start_index_map=(0,))


def _lane_bcast(v16, lane):
    idx = jnp.full((L, 1), lane, dtype=jnp.int32)
    return lax.gather(v16, idx, _BCAST_DNUMS, slice_sizes=(1,),
                      mode=lax.GatherScatterMode.PROMISE_IN_BOUNDS)


def _mm_body(x_ref, w_ref, o_ref):
    o_ref[...] = jnp.dot(x_ref[...], w_ref[...],
                         preferred_element_type=jnp.float32)


def _combine_body(p_ref, o_ref):
    o_ref[...] = jnp.maximum(p_ref[0] + p_ref[1], 0.0)


def _sc_body(h_hbm, src_hbm, dst_hbm, vals_hbm, out_hbm,
             src_v, dst_v, vals_v,
             g0, g1, g2, g3, g4, g5, g6, g7, zb, acc_sh,
             mg0, mg1, mg2, mg3, mg4, mg5, mg6, mg7,
             ms0, ms1, ms2, ms3, ms4, ms5, ms6, ms7, mz):
    cid = lax.axis_index("c")
    sid = lax.axis_index("s")
    wid = cid * NS + sid
    gbufs = (g0, g1, g2, g3, g4, g5, g6, g7)
    gsems = (mg0, mg1, mg2, mg3, mg4, mg5, mg6, mg7)
    ssems = (ms0, ms1, ms2, ms3, ms4, ms5, ms6, ms7)

    def _gather(j, q):
        pltpu.async_copy(h_hbm.at[src_v.at[j]], gbufs[q], gsems[q])

    def _gwait(q):
        pltpu.make_async_copy(h_hbm.at[src_v.at[0]], gbufs[q],
                              gsems[q]).wait()

    def _sfire(j, q):
        pltpu.async_copy(gbufs[q], acc_sh.at[dst_v.at[j]], ssems[q],
                         add=True)

    def _swait(q):
        pltpu.make_async_copy(gbufs[q], acc_sh.at[dst_v.at[0]],
                              ssems[q]).wait()

    def _scale(j, q):
        gb = gbufs[q]
        v16 = vals_v[j, :]

        def _e(lane, c2):
            v = _lane_bcast(v16, lane)
            for t in range(D // L):
                gb[lane, pl.ds(t * L, L)] = gb[lane, pl.ds(t * L, L)] * v
            return c2
        lax.fori_loop(0, C, _e, 0)

    for blk in range(NBLK):
        pltpu.sync_copy(src_hbm.at[wid, pl.ds(blk * NB, NB)], src_v)
        pltpu.sync_copy(dst_hbm.at[wid, pl.ds(blk * NB, NB)], dst_v)
        pltpu.sync_copy(vals_hbm.at[wid, pl.ds(blk * NB, NB)], vals_v)

        for q in range(NG):
            _gather(q, q)

        if blk == 0:
            def _zrow(i, carry):
                for t in range(D // L):
                    zb[i, pl.ds(t * L, L)] = jnp.zeros((L,), jnp.float32)
                return carry
            lax.fori_loop(0, ZR, _zrow, 0)
            for b in range(RPT // ZR):
                pltpu.async_copy(
                    zb, acc_sh.at[pl.ds(sid * RPT + b * ZR, ZR)], mz)
            for b in range(RPT // ZR):
                pltpu.make_async_copy(
                    zb, acc_sh.at[pl.ds(sid * RPT, ZR)], mz).wait()
            plsc.subcore_barrier()

        def _oct(p, carry):
            j0 = NG * p
            for q in range(NG):
                j = j0 + q
                _gwait(q)
                _scale(j, q)
                _sfire(j, q)
                if q >= 2:
                    qq = q - 2

                    @pl.when(p < NO - 1)
                    def _():
                        _swait(qq)
                        _gather(j0 + NG + qq, qq)

            @pl.when(p < NO - 1)
            def _():
                for qq in (NG - 2, NG - 1):
                    _swait(qq)
                    _gather(j0 + NG + qq, qq)
            return carry
        lax.fori_loop(0, NO, _oct, 0)
        for q in range(NG):
            _swait(q)

    plsc.subcore_barrier()
    pltpu.sync_copy(acc_sh.at[pl.ds(sid * RPT, RPT)],
                    out_hbm.at[cid, pl.ds(sid * RPT, RPT)])


_sc_call = functools.partial(
    pl.kernel,
    out_type=jax.ShapeDtypeStruct((NC, NP, D), jnp.float32),
    mesh=plsc.VectorSubcoreMesh(core_axis_name="c", subcore_axis_name="s"),
    compiler_params=pltpu.CompilerParams(use_tc_tiling_on_sc=False),
    scratch_types=(
        [pltpu.VMEM((NB, C), jnp.int32),
         pltpu.VMEM((NB, C), jnp.int32),
         pltpu.VMEM((NB, C), jnp.float32)]
        + [pltpu.VMEM((C, D), jnp.float32) for _ in range(NG)]
        + [pltpu.VMEM((ZR, D), jnp.float32)]
        + [pltpu.VMEM_SHARED((NP, D), jnp.float32)]
        + [pltpu.SemaphoreType.DMA for _ in range(2 * NG + 1)]
    ),
)(_sc_body)


def kernel(x, W, edge_index, adj_vals):
    h = pl.pallas_call(
        _mm_body,
        grid=(5,),
        in_specs=[pl.BlockSpec((N // 5, D), lambda i: (i, 0)),
                  pl.BlockSpec((D, D), lambda i: (0, 0))],
        out_specs=pl.BlockSpec((N // 5, D), lambda i: (i, 0)),
        out_shape=jax.ShapeDtypeStruct((N, D), jnp.float32),
    )(x, W)

    pad = EP - E
    pi = jnp.arange(pad, dtype=jnp.int32)
    src_r = jnp.concatenate([edge_index[1], pi % N]).reshape(NW, NCH, C)
    dst_r = jnp.concatenate(
        [edge_index[0], N + pi % (NP - N)]).reshape(NW, NCH, C)
    vals_r = jnp.concatenate(
        [adj_vals, jnp.zeros((pad,), jnp.float32)]).reshape(NW, NCH, C)

    partials = _sc_call(h, src_r, dst_r, vals_r)

    out = pl.pallas_call(
        _combine_body,
        grid=(5,),
        in_specs=[pl.BlockSpec((NC, N // 5, D), lambda i: (0, i, 0))],
        out_specs=pl.BlockSpec((N // 5, D), lambda i: (i, 0)),
        out_shape=jax.ShapeDtypeStruct((N, D), jnp.float32),
    )(partials)
    return out
